# Initial kernel scaffold; baseline (speedup 1.0000x reference)
import functools
import math

import jax
import jax.numpy as jnp
from jax import lax
from jax.experimental import pallas as pl
from jax.experimental.pallas import tpu as pltpu
from jax.experimental.pallas import tpu_sc as plsc

_NC = 2
_NS = 16
_NW = _NC * _NS
_CH = 128
_D = 128

_mesh = plsc.VectorSubcoreMesh(core_axis_name="c", subcore_axis_name="s")


@functools.lru_cache(maxsize=None)
def _make_deg_kernel(cpt: int, acc_rows: int):
    rpt = acc_rows // _NS

    @functools.partial(
        pl.kernel,
        mesh=_mesh,
        out_type=jax.ShapeDtypeStruct((_NC, acc_rows, 16), jnp.float32),
        scratch_types=[
            pltpu.VMEM((cpt, _CH), jnp.int32),
            pltpu.VMEM((_CH, 16), jnp.float32),
            pltpu.VMEM_SHARED((acc_rows, 16), jnp.float32),
        ],
    )
    def deg_kernel(dst_hbm, ones_hbm, zeros_hbm, out_hbm, idx_v, ones_v, acc):
        c = lax.axis_index("c")
        s = lax.axis_index("s")
        wid = c * _NS + s
        pltpu.sync_copy(ones_hbm, ones_v)
        pltpu.sync_copy(zeros_hbm, acc.at[pl.ds(s * rpt, rpt)])
        pltpu.sync_copy(dst_hbm.at[pl.ds(wid * cpt, cpt)], idx_v)
        plsc.subcore_barrier()

        def chunk(i, carry):
            pltpu.sync_copy(ones_v, acc.at[idx_v.at[i]], add=True)
            return carry

        lax.fori_loop(0, cpt, chunk, 0)
        plsc.subcore_barrier()
        pltpu.sync_copy(acc.at[pl.ds(s * rpt, rpt)],
                        out_hbm.at[c, pl.ds(s * rpt, rpt)])

    return deg_kernel


@functools.lru_cache(maxsize=None)
def _make_seg_kernel(cpt: int, acc_rows: int):
    rpt = acc_rows // _NS

    @functools.partial(
        pl.kernel,
        mesh=_mesh,
        out_type=jax.ShapeDtypeStruct((_NC, acc_rows, _D), jnp.float32),
        scratch_types=[
            pltpu.VMEM((cpt, _CH), jnp.int32),
            pltpu.VMEM((cpt, _CH), jnp.int32),
            pltpu.VMEM((_CH, _D), jnp.float32),
            pltpu.VMEM_SHARED((acc_rows, _D), jnp.float32),
            pltpu.SemaphoreType.DMA,
        ],
    )
    def seg_kernel(hs_hbm, src_hbm, dst_hbm, zeros_hbm, out_hbm,
                   sidx, didx, rows, acc, sem):
        c = lax.axis_index("c")
        s = lax.axis_index("s")
        wid = c * _NS + s
        pltpu.sync_copy(zeros_hbm, acc.at[pl.ds(s * rpt, rpt)])
        pltpu.sync_copy(src_hbm.at[pl.ds(wid * cpt, cpt)], sidx)
        pltpu.sync_copy(dst_hbm.at[pl.ds(wid * cpt, cpt)], didx)
        plsc.subcore_barrier()

        def chunk(i, carry):
            pltpu.async_copy(hs_hbm.at[sidx.at[i]], rows, sem).wait()
            pltpu.sync_copy(rows, acc.at[didx.at[i]], add=True)
            return carry

        lax.fori_loop(0, cpt, chunk, 0)
        plsc.subcore_barrier()
        pltpu.sync_copy(acc.at[pl.ds(s * rpt, rpt)],
                        out_hbm.at[c, pl.ds(s * rpt, rpt)])

    return seg_kernel


def _dinv_from_parts(dp):
    deg = dp[0, :, 0] + dp[1, :, 0] + 1.0
    return lax.rsqrt(deg)


def _stage1_body(x_ref, wp_ref, bp_ref, w_ref, degp_ref, hs_ref):
    dinv = _dinv_from_parts(degp_ref[...])
    x1 = jnp.maximum(
        jnp.dot(x_ref[...], wp_ref[...], preferred_element_type=jnp.float32)
        + bp_ref[...][None, :], 0.0)
    h = jnp.dot(x1, w_ref[...], preferred_element_type=jnp.float32)
    hs_ref[...] = dinv[:, None] * h


def _mid_body(sp_ref, hs_ref, degp_ref, b1_ref, wp2_ref, bp2_ref, w2_ref, hs2_ref):
    dinv = _dinv_from_parts(degp_ref[...])
    sp = sp_ref[...]
    agg = sp[0] + sp[1] + hs_ref[...]
    out1 = jnp.maximum(dinv[:, None] * agg + b1_ref[...][None, :], 0.0)
    x2 = jnp.maximum(
        jnp.dot(out1, wp2_ref[...], preferred_element_type=jnp.float32)
        + bp2_ref[...][None, :], 0.0)
    hs2_ref[...] = dinv[:, None] * jnp.dot(
        x2, w2_ref[...], preferred_element_type=jnp.float32)


def _fin_body(sp_ref, hs_ref, degp_ref, b2_ref, out_ref):
    dinv = _dinv_from_parts(degp_ref[...])
    sp = sp_ref[...]
    agg = sp[0] + sp[1] + hs_ref[...]
    out_ref[...] = jnp.maximum(dinv[:, None] * agg + b2_ref[...][None, :], 0.0)


def _row_spec(br):
    return pl.BlockSpec((br, _D), lambda i: (i, 0))


def _full_mat():
    return pl.BlockSpec((_D, _D), lambda i: (0, 0))


def _full_vec():
    return pl.BlockSpec((_D,), lambda i: (0,))


def _parts_spec(br, minor):
    return pl.BlockSpec((_NC, br, minor), lambda i: (0, i, 0))


def _tc_stage1(x, Wp1, bp1, W1, degp, br):
    n = x.shape[0]
    return pl.pallas_call(
        _stage1_body,
        grid=(n // br,),
        in_specs=[_row_spec(br), _full_mat(), _full_vec(), _full_mat(),
                  _parts_spec(br, 16)],
        out_specs=_row_spec(br),
        out_shape=jax.ShapeDtypeStruct((n, _D), jnp.float32),
    )(x, Wp1, bp1, W1, degp)


def _tc_mid(s1, hs1, degp, b1, Wp2, bp2, W2, br):
    n = hs1.shape[0]
    return pl.pallas_call(
        _mid_body,
        grid=(n // br,),
        in_specs=[_parts_spec(br, _D), _row_spec(br), _parts_spec(br, 16),
                  _full_vec(), _full_mat(), _full_vec(), _full_mat()],
        out_specs=_row_spec(br),
        out_shape=jax.ShapeDtypeStruct((n, _D), jnp.float32),
    )(s1, hs1, degp, b1, Wp2, bp2, W2)


def _tc_fin(s2, hs2, degp, b2, br):
    n = hs2.shape[0]
    return pl.pallas_call(
        _fin_body,
        grid=(n // br,),
        in_specs=[_parts_spec(br, _D), _row_spec(br), _parts_spec(br, 16),
                  _full_vec()],
        out_specs=_row_spec(br),
        out_shape=jax.ShapeDtypeStruct((n, _D), jnp.float32),
    )(s2, hs2, degp, b2)


def kernel(x, edge_index, Wp1, bp1, W1, b1, Wp2, bp2, W2, b2):
    n = x.shape[0]
    e = edge_index.shape[1]
    cpt = -(-e // (_NW * _CH))
    e_pad = cpt * _NW * _CH
    acc_rows = math.ceil((n + 1) / 128) * 128
    rpt = acc_rows // _NS
    br = 2000 if n % 2000 == 0 else 8 * math.gcd(n // 8, 1000)

    src = edge_index[0]
    dst = edge_index[1]
    pad = e_pad - e
    if pad:
        src = jnp.concatenate([src, jnp.zeros((pad,), src.dtype)])
        dst = jnp.concatenate([dst, jnp.full((pad,), n, dst.dtype)])
    src2d = src.reshape(_NW * cpt, _CH)
    dst2d = dst.reshape(_NW * cpt, _CH)

    ones16 = jnp.ones((_CH, 16), jnp.float32)
    zdeg = jnp.zeros((rpt, 16), jnp.float32)
    zseg = jnp.zeros((rpt, _D), jnp.float32)

    deg_k = _make_deg_kernel(cpt, acc_rows)
    seg_k = _make_seg_kernel(cpt, acc_rows)

    degp = deg_k(dst2d, ones16, zdeg)
    hs1 = _tc_stage1(x, Wp1, bp1, W1, degp, br)
    s1 = seg_k(hs1, src2d, dst2d, zseg)
    hs2 = _tc_mid(s1, hs1, degp, b1, Wp2, bp2, W2, br)
    s2 = seg_k(hs2, src2d, dst2d, zseg)
    return _tc_fin(s2, hs2, degp, b2, br)


# speedup vs baseline: 8.9056x; 8.9056x over previous
import functools
import math

import jax
import jax.numpy as jnp
from jax import lax
from jax.experimental import pallas as pl
from jax.experimental.pallas import tpu as pltpu
from jax.experimental.pallas import tpu_sc as plsc

_NC = 2
_NS = 16
_NW = _NC * _NS
_CH = 128
_D = 128

_mesh = plsc.VectorSubcoreMesh(core_axis_name="c", subcore_axis_name="s")


@functools.lru_cache(maxsize=None)
def _make_deg_kernel(cpt: int, acc_rows: int):
    rpt = acc_rows // _NS

    @functools.partial(
        pl.kernel,
        mesh=_mesh,
        out_type=jax.ShapeDtypeStruct((_NC, acc_rows, _D), jnp.float32),
        scratch_types=[
            pltpu.VMEM((cpt, _CH), jnp.int32),
            pltpu.VMEM((_CH, _D), jnp.float32),
            pltpu.VMEM_SHARED((acc_rows, _D), jnp.float32),
        ],
    )
    def deg_kernel(dst_hbm, ones_hbm, zeros_hbm, out_hbm, idx_v, ones_v, acc):
        c = lax.axis_index("c")
        s = lax.axis_index("s")
        wid = c * _NS + s
        pltpu.sync_copy(ones_hbm, ones_v)
        pltpu.sync_copy(zeros_hbm, acc.at[pl.ds(s * rpt, rpt)])
        pltpu.sync_copy(dst_hbm.at[pl.ds(wid * cpt, cpt)], idx_v)
        plsc.subcore_barrier()

        def chunk(i, carry):
            pltpu.sync_copy(ones_v, acc.at[idx_v.at[i]], add=True)
            return carry

        lax.fori_loop(0, cpt, chunk, 0)
        plsc.subcore_barrier()
        pltpu.sync_copy(acc.at[pl.ds(s * rpt, rpt)],
                        out_hbm.at[c, pl.ds(s * rpt, rpt)])

    return deg_kernel


@functools.lru_cache(maxsize=None)
def _make_seg_kernel(cpt: int, acc_rows: int):
    rpt = acc_rows // _NS

    @functools.partial(
        pl.kernel,
        mesh=_mesh,
        out_type=jax.ShapeDtypeStruct((_NC, acc_rows, _D), jnp.float32),
        scratch_types=[
            pltpu.VMEM((cpt, _CH), jnp.int32),
            pltpu.VMEM((cpt, _CH), jnp.int32),
            pltpu.VMEM((_CH, _D), jnp.float32),
            pltpu.VMEM_SHARED((acc_rows, _D), jnp.float32),
            pltpu.SemaphoreType.DMA,
        ],
    )
    def seg_kernel(hs_hbm, src_hbm, dst_hbm, zeros_hbm, out_hbm,
                   sidx, didx, rows, acc, sem):
        c = lax.axis_index("c")
        s = lax.axis_index("s")
        wid = c * _NS + s
        pltpu.sync_copy(zeros_hbm, acc.at[pl.ds(s * rpt, rpt)])
        pltpu.sync_copy(src_hbm.at[pl.ds(wid * cpt, cpt)], sidx)
        pltpu.sync_copy(dst_hbm.at[pl.ds(wid * cpt, cpt)], didx)
        plsc.subcore_barrier()

        def chunk(i, carry):
            pltpu.async_copy(hs_hbm.at[sidx.at[i]], rows, sem).wait()
            pltpu.sync_copy(rows, acc.at[didx.at[i]], add=True)
            return carry

        lax.fori_loop(0, cpt, chunk, 0)
        plsc.subcore_barrier()
        pltpu.sync_copy(acc.at[pl.ds(s * rpt, rpt)],
                        out_hbm.at[c, pl.ds(s * rpt, rpt)])

    return seg_kernel


def _dinv_from_parts(dp):
    deg = dp[0, :, 0] + dp[1, :, 0] + 1.0
    return lax.rsqrt(deg)


def _stage1_body(x_ref, wp_ref, bp_ref, w_ref, degp_ref, hs_ref):
    dinv = _dinv_from_parts(degp_ref[...])
    x1 = jnp.maximum(
        jnp.dot(x_ref[...], wp_ref[...], preferred_element_type=jnp.float32)
        + bp_ref[...][None, :], 0.0)
    h = jnp.dot(x1, w_ref[...], preferred_element_type=jnp.float32)
    hs_ref[...] = dinv[:, None] * h


def _mid_body(sp_ref, hs_ref, degp_ref, b1_ref, wp2_ref, bp2_ref, w2_ref, hs2_ref):
    dinv = _dinv_from_parts(degp_ref[...])
    sp = sp_ref[...]
    agg = sp[0] + sp[1] + hs_ref[...]
    out1 = jnp.maximum(dinv[:, None] * agg + b1_ref[...][None, :], 0.0)
    x2 = jnp.maximum(
        jnp.dot(out1, wp2_ref[...], preferred_element_type=jnp.float32)
        + bp2_ref[...][None, :], 0.0)
    hs2_ref[...] = dinv[:, None] * jnp.dot(
        x2, w2_ref[...], preferred_element_type=jnp.float32)


def _fin_body(sp_ref, hs_ref, degp_ref, b2_ref, out_ref):
    dinv = _dinv_from_parts(degp_ref[...])
    sp = sp_ref[...]
    agg = sp[0] + sp[1] + hs_ref[...]
    out_ref[...] = jnp.maximum(dinv[:, None] * agg + b2_ref[...][None, :], 0.0)


def _row_spec(br):
    return pl.BlockSpec((br, _D), lambda i: (i, 0))


def _full_mat():
    return pl.BlockSpec((_D, _D), lambda i: (0, 0))


def _full_vec():
    return pl.BlockSpec((_D,), lambda i: (0,))


def _parts_spec(br, minor):
    return pl.BlockSpec((_NC, br, minor), lambda i: (0, i, 0))


def _tc_stage1(x, Wp1, bp1, W1, degp, br):
    n = x.shape[0]
    return pl.pallas_call(
        _stage1_body,
        grid=(n // br,),
        in_specs=[_row_spec(br), _full_mat(), _full_vec(), _full_mat(),
                  _parts_spec(br, _D)],
        out_specs=_row_spec(br),
        out_shape=jax.ShapeDtypeStruct((n, _D), jnp.float32),
    )(x, Wp1, bp1, W1, degp)


def _tc_mid(s1, hs1, degp, b1, Wp2, bp2, W2, br):
    n = hs1.shape[0]
    return pl.pallas_call(
        _mid_body,
        grid=(n // br,),
        in_specs=[_parts_spec(br, _D), _row_spec(br), _parts_spec(br, _D),
                  _full_vec(), _full_mat(), _full_vec(), _full_mat()],
        out_specs=_row_spec(br),
        out_shape=jax.ShapeDtypeStruct((n, _D), jnp.float32),
    )(s1, hs1, degp, b1, Wp2, bp2, W2)


def _tc_fin(s2, hs2, degp, b2, br):
    n = hs2.shape[0]
    return pl.pallas_call(
        _fin_body,
        grid=(n // br,),
        in_specs=[_parts_spec(br, _D), _row_spec(br), _parts_spec(br, _D),
                  _full_vec()],
        out_specs=_row_spec(br),
        out_shape=jax.ShapeDtypeStruct((n, _D), jnp.float32),
    )(s2, hs2, degp, b2)


def kernel(x, edge_index, Wp1, bp1, W1, b1, Wp2, bp2, W2, b2):
    n = x.shape[0]
    e = edge_index.shape[1]
    cpt = -(-e // (_NW * _CH))
    cpt = ((cpt + 7) // 8) * 8
    e_pad = cpt * _NW * _CH
    acc_rows = math.ceil((n + 1) / 128) * 128
    rpt = acc_rows // _NS
    br = 2000 if n % 2000 == 0 else 8 * math.gcd(n // 8, 1000)

    src = edge_index[0]
    dst = edge_index[1]
    pad = e_pad - e
    if pad:
        src = jnp.concatenate([src, jnp.zeros((pad,), src.dtype)])
        dst = jnp.concatenate([dst, jnp.full((pad,), n, dst.dtype)])
    src2d = src.reshape(_NW * cpt, _CH)
    dst2d = dst.reshape(_NW * cpt, _CH)

    ones_rows = jnp.ones((_CH, _D), jnp.float32)
    zseg = jnp.zeros((rpt, _D), jnp.float32)

    deg_k = _make_deg_kernel(cpt, acc_rows)
    seg_k = _make_seg_kernel(cpt, acc_rows)

    degp = deg_k(dst2d, ones_rows, zseg)
    hs1 = _tc_stage1(x, Wp1, bp1, W1, degp, br)
    s1 = seg_k(hs1, src2d, dst2d, zseg)
    hs2 = _tc_mid(s1, hs1, degp, b1, Wp2, bp2, W2, br)
    s2 = seg_k(hs2, src2d, dst2d, zseg)
    return _tc_fin(s2, hs2, degp, b2, br)


# speedup vs baseline: 10.2373x; 1.1495x over previous
import functools
import math

import jax
import jax.numpy as jnp
from jax import lax
from jax.experimental import pallas as pl
from jax.experimental.pallas import tpu as pltpu
from jax.experimental.pallas import tpu_sc as plsc

_NC = 2
_NS = 16
_NW = _NC * _NS
_CH = 128
_D = 128
_NB = 4

_mesh = plsc.VectorSubcoreMesh(core_axis_name="c", subcore_axis_name="s")


@functools.lru_cache(maxsize=None)
def _make_deg_kernel(cpt: int, acc_rows: int):
    rpt = acc_rows // _NS

    @functools.partial(
        pl.kernel,
        mesh=_mesh,
        out_type=jax.ShapeDtypeStruct((_NC, acc_rows, _D), jnp.float32),
        scratch_types=[
            pltpu.VMEM((cpt, _CH), jnp.int32),
            pltpu.VMEM((_CH, _D), jnp.float32),
            pltpu.VMEM_SHARED((acc_rows, _D), jnp.float32),
            pltpu.SemaphoreType.DMA,
        ],
    )
    def deg_kernel(dst_hbm, ones_hbm, zeros_hbm, out_hbm, idx_v, ones_v, acc, sem):
        c = lax.axis_index("c")
        s = lax.axis_index("s")
        wid = c * _NS + s
        pltpu.sync_copy(ones_hbm, ones_v)
        pltpu.sync_copy(zeros_hbm, acc.at[pl.ds(s * rpt, rpt)])
        pltpu.sync_copy(dst_hbm.at[pl.ds(wid * cpt, cpt)], idx_v)
        plsc.subcore_barrier()

        def chunk(i, carry):
            pltpu.async_copy(ones_v, acc.at[idx_v.at[i]], sem, add=True)
            return carry

        lax.fori_loop(0, cpt, chunk, 0)

        def drain(i, carry):
            pltpu.make_async_copy(ones_v, acc.at[idx_v.at[0]], sem).wait()
            return carry

        lax.fori_loop(0, cpt, drain, 0)
        plsc.subcore_barrier()
        pltpu.sync_copy(acc.at[pl.ds(s * rpt, rpt)],
                        out_hbm.at[c, pl.ds(s * rpt, rpt)])

    return deg_kernel


@functools.lru_cache(maxsize=None)
def _make_seg_kernel(cpt: int, acc_rows: int):
    rpt = acc_rows // _NS

    @functools.partial(
        pl.kernel,
        mesh=_mesh,
        out_type=jax.ShapeDtypeStruct((_NC, acc_rows, _D), jnp.float32),
        scratch_types=[
            pltpu.VMEM((4, _CH), jnp.int32),
            pltpu.VMEM((3, _CH), jnp.int32),
            pltpu.VMEM((3, _CH, _D), jnp.float32),
            pltpu.VMEM_SHARED((acc_rows, _D), jnp.float32),
            pltpu.SemaphoreType.DMA,
            pltpu.SemaphoreType.DMA,
            pltpu.SemaphoreType.DMA,
            pltpu.SemaphoreType.DMA,
        ],
    )
    def seg_kernel(hs_hbm, src_hbm, dst_hbm, zeros_hbm, out_hbm,
                   sidx, didx, rows, acc, semg, sems, semis, semid):
        c = lax.axis_index("c")
        s = lax.axis_index("s")
        wid = c * _NS + s
        base = wid * cpt
        pltpu.sync_copy(zeros_hbm, acc.at[pl.ds(s * rpt, rpt)])
        plsc.subcore_barrier()

        def g_wait(i):
            pltpu.make_async_copy(
                hs_hbm.at[sidx.at[i % 4]], rows.at[i % 3], semg).wait()

        def s_drain(i):
            pltpu.make_async_copy(
                rows.at[i % 3], acc.at[didx.at[i % 3]], sems).wait()

        for k in range(4):
            pltpu.async_copy(src_hbm.at[base + k], sidx.at[k], semis)
        for k in range(2):
            pltpu.async_copy(dst_hbm.at[base + k], didx.at[k], semid)
        for k in range(2):
            pltpu.make_async_copy(src_hbm.at[base], sidx.at[k], semis).wait()
            pltpu.async_copy(hs_hbm.at[sidx.at[k]], rows.at[k], semg)

        def body(i, carry):
            pltpu.make_async_copy(dst_hbm.at[base], didx.at[i % 3], semid).wait()
            g_wait(i)
            pltpu.async_copy(rows.at[i % 3], acc.at[didx.at[i % 3]], sems,
                             add=True)

            @pl.when(i + 4 < cpt)
            def _():
                pltpu.async_copy(src_hbm.at[base + i + 4], sidx.at[i % 4], semis)

            @pl.when(i >= 1)
            def _():
                s_drain(i - 1)

            @pl.when(i + 2 < cpt)
            def _():
                pltpu.async_copy(dst_hbm.at[base + i + 2], didx.at[(i + 2) % 3],
                                 semid)
                pltpu.make_async_copy(src_hbm.at[base], sidx.at[(i + 2) % 4],
                                      semis).wait()
                pltpu.async_copy(hs_hbm.at[sidx.at[(i + 2) % 4]],
                                 rows.at[(i + 2) % 3], semg)

            return carry

        lax.fori_loop(0, cpt, body, 0)
        s_drain(cpt - 1)
        plsc.subcore_barrier()
        pltpu.sync_copy(acc.at[pl.ds(s * rpt, rpt)],
                        out_hbm.at[c, pl.ds(s * rpt, rpt)])

    return seg_kernel


def _dinv_from_parts(dp):
    deg = dp[0, :, 0] + dp[1, :, 0] + 1.0
    return lax.rsqrt(deg)


def _stage1_body(x_ref, wp_ref, bp_ref, w_ref, degp_ref, hs_ref):
    dinv = _dinv_from_parts(degp_ref[...])
    x1 = jnp.maximum(
        jnp.dot(x_ref[...], wp_ref[...], preferred_element_type=jnp.float32)
        + bp_ref[...][None, :], 0.0)
    h = jnp.dot(x1, w_ref[...], preferred_element_type=jnp.float32)
    hs_ref[...] = dinv[:, None] * h


def _mid_body(sp_ref, hs_ref, degp_ref, b1_ref, wp2_ref, bp2_ref, w2_ref, hs2_ref):
    dinv = _dinv_from_parts(degp_ref[...])
    sp = sp_ref[...]
    agg = sp[0] + sp[1] + hs_ref[...]
    out1 = jnp.maximum(dinv[:, None] * agg + b1_ref[...][None, :], 0.0)
    x2 = jnp.maximum(
        jnp.dot(out1, wp2_ref[...], preferred_element_type=jnp.float32)
        + bp2_ref[...][None, :], 0.0)
    hs2_ref[...] = dinv[:, None] * jnp.dot(
        x2, w2_ref[...], preferred_element_type=jnp.float32)


def _fin_body(sp_ref, hs_ref, degp_ref, b2_ref, out_ref):
    dinv = _dinv_from_parts(degp_ref[...])
    sp = sp_ref[...]
    agg = sp[0] + sp[1] + hs_ref[...]
    out_ref[...] = jnp.maximum(dinv[:, None] * agg + b2_ref[...][None, :], 0.0)


def _row_spec(br):
    return pl.BlockSpec((br, _D), lambda i: (i, 0))


def _full_mat():
    return pl.BlockSpec((_D, _D), lambda i: (0, 0))


def _full_vec():
    return pl.BlockSpec((_D,), lambda i: (0,))


def _parts_spec(br, minor):
    return pl.BlockSpec((_NC, br, minor), lambda i: (0, i, 0))


def _tc_stage1(x, Wp1, bp1, W1, degp, br):
    n = x.shape[0]
    return pl.pallas_call(
        _stage1_body,
        grid=(n // br,),
        in_specs=[_row_spec(br), _full_mat(), _full_vec(), _full_mat(),
                  _parts_spec(br, _D)],
        out_specs=_row_spec(br),
        out_shape=jax.ShapeDtypeStruct((n, _D), jnp.float32),
    )(x, Wp1, bp1, W1, degp)


def _tc_mid(s1, hs1, degp, b1, Wp2, bp2, W2, br):
    n = hs1.shape[0]
    return pl.pallas_call(
        _mid_body,
        grid=(n // br,),
        in_specs=[_parts_spec(br, _D), _row_spec(br), _parts_spec(br, _D),
                  _full_vec(), _full_mat(), _full_vec(), _full_mat()],
        out_specs=_row_spec(br),
        out_shape=jax.ShapeDtypeStruct((n, _D), jnp.float32),
    )(s1, hs1, degp, b1, Wp2, bp2, W2)


def _tc_fin(s2, hs2, degp, b2, br):
    n = hs2.shape[0]
    return pl.pallas_call(
        _fin_body,
        grid=(n // br,),
        in_specs=[_parts_spec(br, _D), _row_spec(br), _parts_spec(br, _D),
                  _full_vec()],
        out_specs=_row_spec(br),
        out_shape=jax.ShapeDtypeStruct((n, _D), jnp.float32),
    )(s2, hs2, degp, b2)


def kernel(x, edge_index, Wp1, bp1, W1, b1, Wp2, bp2, W2, b2):
    n = x.shape[0]
    e = edge_index.shape[1]
    cpt = -(-e // (_NW * _CH))
    cpt = ((cpt + 7) // 8) * 8
    e_pad = cpt * _NW * _CH
    acc_rows = math.ceil((n + 1) / 128) * 128
    rpt = acc_rows // _NS
    br = 2000 if n % 2000 == 0 else 8 * math.gcd(n // 8, 1000)

    src = edge_index[0]
    dst = edge_index[1]
    pad = e_pad - e
    if pad:
        src = jnp.concatenate([src, jnp.zeros((pad,), src.dtype)])
        dst = jnp.concatenate([dst, jnp.full((pad,), n, dst.dtype)])
    src2d = src.reshape(_NW * cpt, _CH)
    dst2d = dst.reshape(_NW * cpt, _CH)

    ones_rows = jnp.ones((_CH, _D), jnp.float32)
    zseg = jnp.zeros((rpt, _D), jnp.float32)

    deg_k = _make_deg_kernel(cpt, acc_rows)
    seg_k = _make_seg_kernel(cpt, acc_rows)

    degp = deg_k(dst2d, ones_rows, zseg)
    hs1 = _tc_stage1(x, Wp1, bp1, W1, degp, br)
    s1 = seg_k(hs1, src2d, dst2d, zseg)
    hs2 = _tc_mid(s1, hs1, degp, b1, Wp2, bp2, W2, br)
    s2 = seg_k(hs2, src2d, dst2d, zseg)
    return _tc_fin(s2, hs2, degp, b2, br)


# speedup vs baseline: 30.1689x; 2.9469x over previous
import functools
import math

import jax
import jax.numpy as jnp
from jax import lax
from jax.experimental import pallas as pl
from jax.experimental.pallas import tpu as pltpu
from jax.experimental.pallas import tpu_sc as plsc

_NC = 2
_NS = 16
_NW = _NC * _NS
_CH = 128
_D = 128
_SPLIT0 = 80

_mesh = plsc.VectorSubcoreMesh(core_axis_name="c", subcore_axis_name="s")


@functools.lru_cache(maxsize=None)
def _make_deg_kernel(cpt: int, acc_rows: int):
    rpt = acc_rows // _NS

    @functools.partial(
        pl.kernel,
        mesh=_mesh,
        out_type=jax.ShapeDtypeStruct((_NC, acc_rows, _D), jnp.float32),
        scratch_types=[
            pltpu.VMEM((cpt, _CH), jnp.int32),
            pltpu.VMEM((_CH, _D), jnp.float32),
            pltpu.VMEM_SHARED((acc_rows, _D), jnp.float32),
            pltpu.SemaphoreType.DMA,
        ],
    )
    def deg_kernel(dst_hbm, ones_hbm, zeros_hbm, out_hbm, idx_v, ones_v, acc, sem):
        c = lax.axis_index("c")
        s = lax.axis_index("s")
        wid = c * _NS + s
        pltpu.sync_copy(ones_hbm, ones_v)
        pltpu.sync_copy(zeros_hbm, acc.at[pl.ds(s * rpt, rpt)])
        pltpu.sync_copy(dst_hbm.at[pl.ds(wid * cpt, cpt)], idx_v)
        plsc.subcore_barrier()

        def chunk(i, carry):
            pltpu.async_copy(ones_v, acc.at[idx_v.at[i]], sem, add=True)
            return carry

        lax.fori_loop(0, cpt, chunk, 0)

        def drain(i, carry):
            pltpu.make_async_copy(ones_v, acc.at[idx_v.at[0]], sem).wait()
            return carry

        lax.fori_loop(0, cpt, drain, 0)
        plsc.subcore_barrier()
        pltpu.sync_copy(acc.at[pl.ds(s * rpt, rpt)],
                        out_hbm.at[c, pl.ds(s * rpt, rpt)])

    return deg_kernel


@functools.lru_cache(maxsize=None)
def _make_seg_kernel(cpt0: int, cpt1: int, acc_rows: int):
    rpt = acc_rows // _NS

    @functools.partial(
        pl.kernel,
        mesh=_mesh,
        out_type=jax.ShapeDtypeStruct((_NC, acc_rows, _D), jnp.float32),
        scratch_types=[
            pltpu.VMEM((4, _CH), jnp.int32),
            pltpu.VMEM((3, _CH), jnp.int32),
            pltpu.VMEM((3, _CH, _D), jnp.float32),
            pltpu.VMEM_SHARED((acc_rows, _D), jnp.float32),
            pltpu.SemaphoreType.DMA,
            pltpu.SemaphoreType.DMA,
            pltpu.SemaphoreType.DMA,
            pltpu.SemaphoreType.DMA,
        ],
    )
    def seg_kernel(hs0_hbm, hs1_hbm, src_hbm, dst_hbm, zeros_hbm, out_hbm,
                   sidx, didx, rows, acc, semg, sems, semis, semid):
        c = lax.axis_index("c")
        s = lax.axis_index("s")
        pltpu.sync_copy(zeros_hbm, acc.at[pl.ds(s * rpt, rpt)])
        plsc.subcore_barrier()

        def run(hs_hbm, base, n):
            def g_wait(i):
                pltpu.make_async_copy(
                    hs_hbm.at[sidx.at[i % 4]], rows.at[i % 3], semg).wait()

            def s_drain(i):
                pltpu.make_async_copy(
                    rows.at[i % 3], acc.at[didx.at[i % 3]], sems).wait()

            for k in range(4):
                pltpu.async_copy(src_hbm.at[base + k], sidx.at[k], semis)
            for k in range(2):
                pltpu.async_copy(dst_hbm.at[base + k], didx.at[k], semid)
            for k in range(2):
                pltpu.make_async_copy(src_hbm.at[base], sidx.at[k], semis).wait()
                pltpu.async_copy(hs_hbm.at[sidx.at[k]], rows.at[k], semg)

            def body(i, carry):
                pltpu.make_async_copy(dst_hbm.at[base], didx.at[i % 3],
                                      semid).wait()
                g_wait(i)
                pltpu.async_copy(rows.at[i % 3], acc.at[didx.at[i % 3]], sems,
                                 add=True)

                @pl.when(i + 4 < n)
                def _():
                    pltpu.async_copy(src_hbm.at[base + i + 4], sidx.at[i % 4],
                                     semis)

                @pl.when(i >= 1)
                def _():
                    s_drain(i - 1)

                @pl.when(i + 2 < n)
                def _():
                    pltpu.async_copy(dst_hbm.at[base + i + 2],
                                     didx.at[(i + 2) % 3], semid)
                    pltpu.make_async_copy(src_hbm.at[base], sidx.at[(i + 2) % 4],
                                          semis).wait()
                    pltpu.async_copy(hs_hbm.at[sidx.at[(i + 2) % 4]],
                                     rows.at[(i + 2) % 3], semg)

                return carry

            lax.fori_loop(0, n, body, 0)
            s_drain(n - 1)

        @pl.when(c == 0)
        def _():
            run(hs0_hbm, s * cpt0, cpt0)

        @pl.when(c == 1)
        def _():
            run(hs1_hbm, _NS * cpt0 + s * cpt1, cpt1)

        plsc.subcore_barrier()
        pltpu.sync_copy(acc.at[pl.ds(s * rpt, rpt)],
                        out_hbm.at[c, pl.ds(s * rpt, rpt)])

    return seg_kernel


def _dinv_from_parts(dp):
    deg = dp[0, :, 0] + dp[1, :, 0] + 1.0
    return lax.rsqrt(deg)


def _stage1_body(x_ref, wp_ref, bp_ref, w_ref, degp_ref, hs_ref):
    dinv = _dinv_from_parts(degp_ref[...])
    x1 = jnp.maximum(
        jnp.dot(x_ref[...], wp_ref[...], preferred_element_type=jnp.float32)
        + bp_ref[...][None, :], 0.0)
    h = jnp.dot(x1, w_ref[...], preferred_element_type=jnp.float32)
    hs_ref[...] = dinv[:, None] * h


def _mid_body(sp_ref, hs_ref, degp_ref, b1_ref, wp2_ref, bp2_ref, w2_ref, hs2_ref):
    dinv = _dinv_from_parts(degp_ref[...])
    sp = sp_ref[...]
    agg = sp[0] + sp[1] + hs_ref[...]
    out1 = jnp.maximum(dinv[:, None] * agg + b1_ref[...][None, :], 0.0)
    x2 = jnp.maximum(
        jnp.dot(out1, wp2_ref[...], preferred_element_type=jnp.float32)
        + bp2_ref[...][None, :], 0.0)
    hs2_ref[...] = dinv[:, None] * jnp.dot(
        x2, w2_ref[...], preferred_element_type=jnp.float32)


def _fin_body(sp_ref, hs_ref, degp_ref, b2_ref, out_ref):
    dinv = _dinv_from_parts(degp_ref[...])
    sp = sp_ref[...]
    agg = sp[0] + sp[1] + hs_ref[...]
    out_ref[...] = jnp.maximum(dinv[:, None] * agg + b2_ref[...][None, :], 0.0)


def _row_spec(br):
    return pl.BlockSpec((br, _D), lambda i: (i, 0))


def _full_mat():
    return pl.BlockSpec((_D, _D), lambda i: (0, 0))


def _full_vec():
    return pl.BlockSpec((_D,), lambda i: (0,))


def _parts_spec(br, minor):
    return pl.BlockSpec((_NC, br, minor), lambda i: (0, i, 0))


def _tc_stage1(x, Wp1, bp1, W1, degp, br):
    n = x.shape[0]
    return pl.pallas_call(
        _stage1_body,
        grid=(n // br,),
        in_specs=[_row_spec(br), _full_mat(), _full_vec(), _full_mat(),
                  _parts_spec(br, _D)],
        out_specs=_row_spec(br),
        out_shape=jax.ShapeDtypeStruct((n, _D), jnp.float32),
    )(x, Wp1, bp1, W1, degp)


def _tc_mid(s1, hs1, degp, b1, Wp2, bp2, W2, br):
    n = hs1.shape[0]
    return pl.pallas_call(
        _mid_body,
        grid=(n // br,),
        in_specs=[_parts_spec(br, _D), _row_spec(br), _parts_spec(br, _D),
                  _full_vec(), _full_mat(), _full_vec(), _full_mat()],
        out_specs=_row_spec(br),
        out_shape=jax.ShapeDtypeStruct((n, _D), jnp.float32),
    )(s1, hs1, degp, b1, Wp2, bp2, W2)


def _tc_fin(s2, hs2, degp, b2, br):
    n = hs2.shape[0]
    return pl.pallas_call(
        _fin_body,
        grid=(n // br,),
        in_specs=[_parts_spec(br, _D), _row_spec(br), _parts_spec(br, _D),
                  _full_vec()],
        out_specs=_row_spec(br),
        out_shape=jax.ShapeDtypeStruct((n, _D), jnp.float32),
    )(s2, hs2, degp, b2)


def kernel(x, edge_index, Wp1, bp1, W1, b1, Wp2, bp2, W2, b2):
    n = x.shape[0]
    e = edge_index.shape[1]
    cpt = -(-e // (_NW * _CH))
    cpt = ((cpt + 7) // 8) * 8
    e_pad = cpt * _NW * _CH
    acc_rows = math.ceil((n + 1) / 128) * 128
    rpt = acc_rows // _NS
    br = 2000 if n % 2000 == 0 else 8 * math.gcd(n // 8, 1000)

    src = edge_index[0]
    dst = edge_index[1]
    pad = e_pad - e
    if pad:
        pi = jnp.arange(pad, dtype=src.dtype)
        src = jnp.concatenate([src, pi % n])
        dst = jnp.concatenate([dst, n + pi % (acc_rows - n)])
    src2d = src.reshape(_NW * cpt, _CH)
    dst2d = dst.reshape(_NW * cpt, _CH)

    ones_rows = jnp.ones((_CH, _D), jnp.float32)
    zseg = jnp.zeros((rpt, _D), jnp.float32)

    cpt2 = 2 * cpt
    cpt0 = _SPLIT0 if _SPLIT0 < cpt2 else cpt
    cpt1 = cpt2 - cpt0
    deg_k = _make_deg_kernel(cpt, acc_rows)
    seg_k = _make_seg_kernel(cpt0, cpt1, acc_rows)

    degp = deg_k(dst2d, ones_rows, zseg)
    hs1 = _tc_stage1(x, Wp1, bp1, W1, degp, br)
    s1 = seg_k(hs1, hs1, src2d, dst2d, zseg)
    hs2 = _tc_mid(s1, hs1, degp, b1, Wp2, bp2, W2, br)
    s2 = seg_k(hs2, hs2, src2d, dst2d, zseg)
    return _tc_fin(s2, hs2, degp, b2, br)


# speedup vs baseline: 31.7041x; 1.0509x over previous
import functools
import math

import jax
import jax.numpy as jnp
from jax import lax
from jax.experimental import pallas as pl
from jax.experimental.pallas import tpu as pltpu
from jax.experimental.pallas import tpu_sc as plsc

_NC = 2
_NS = 16
_NW = _NC * _NS
_CH = 128
_D = 128

_mesh = plsc.VectorSubcoreMesh(core_axis_name="c", subcore_axis_name="s",
                               num_cores=_NC, num_subcores=_NS)


@functools.lru_cache(maxsize=None)
def _make_deg_kernel(e_chunks: int, e_off: int, acc_rows: int):
    rpt = acc_rows // _NS
    q, r = divmod(e_chunks, _NW)

    @functools.partial(
        pl.kernel,
        mesh=_mesh,
        out_type=jax.ShapeDtypeStruct((_NC, acc_rows, _D), jnp.float32),
        scratch_types=[
            pltpu.VMEM((q + 1, _CH), jnp.int32),
            pltpu.VMEM((_CH, _D), jnp.float32),
            pltpu.VMEM_SHARED((acc_rows, _D), jnp.float32),
            pltpu.SemaphoreType.DMA,
            pltpu.SemaphoreType.DMA,
        ],
    )
    def deg_kernel(ei_hbm, ones_hbm, zeros_hbm, out_hbm, idx_v, ones_v, acc,
                   semi, sem):
        c = lax.axis_index("c")
        s = lax.axis_index("s")
        wid = c * _NS + s
        n = q + jnp.where(wid < r, 1, 0)
        base = q * wid + jnp.minimum(wid, r)
        pltpu.sync_copy(ones_hbm, ones_v)
        pltpu.sync_copy(zeros_hbm, acc.at[pl.ds(s * rpt, rpt)])

        def load(k, carry):
            pltpu.async_copy(
                ei_hbm.at[pl.ds(e_off + (base + k) * _CH, _CH)],
                idx_v.at[k], semi)
            return carry

        lax.fori_loop(0, n, load, 0)

        def drain_i(k, carry):
            pltpu.make_async_copy(ei_hbm.at[pl.ds(e_off, _CH)], idx_v.at[0],
                                  semi).wait()
            return carry

        lax.fori_loop(0, n, drain_i, 0)
        plsc.subcore_barrier()

        def chunk(i, carry):
            pltpu.async_copy(ones_v, acc.at[idx_v.at[i]], sem, add=True)
            return carry

        lax.fori_loop(0, n, chunk, 0)

        def drain(i, carry):
            pltpu.make_async_copy(ones_v, acc.at[idx_v.at[0]], sem).wait()
            return carry

        lax.fori_loop(0, n, drain, 0)
        plsc.subcore_barrier()
        pltpu.sync_copy(acc.at[pl.ds(s * rpt, rpt)],
                        out_hbm.at[c, pl.ds(s * rpt, rpt)])

    return deg_kernel


@functools.lru_cache(maxsize=None)
def _make_seg_kernel(e_chunks: int, e_off: int, acc_rows: int):
    rpt = acc_rows // _NS
    q, r = divmod(e_chunks, _NW)

    @functools.partial(
        pl.kernel,
        mesh=_mesh,
        out_type=jax.ShapeDtypeStruct((_NC, acc_rows, _D), jnp.float32),
        scratch_types=[
            pltpu.VMEM((4, _CH), jnp.int32),
            pltpu.VMEM((3, _CH), jnp.int32),
            pltpu.VMEM((3, _CH, _D), jnp.float32),
            pltpu.VMEM_SHARED((acc_rows, _D), jnp.float32),
            pltpu.SemaphoreType.DMA,
            pltpu.SemaphoreType.DMA,
            pltpu.SemaphoreType.DMA,
            pltpu.SemaphoreType.DMA,
        ],
    )
    def seg_kernel(hs_hbm, ei_hbm, zeros_hbm, out_hbm,
                   sidx, didx, rows, acc, semg, sems, semis, semid):
        c = lax.axis_index("c")
        s = lax.axis_index("s")
        wid = c * _NS + s
        n = q + jnp.where(wid < r, 1, 0)
        base = q * wid + jnp.minimum(wid, r)
        pltpu.sync_copy(zeros_hbm, acc.at[pl.ds(s * rpt, rpt)])
        plsc.subcore_barrier()

        def s_row(k):
            return ei_hbm.at[pl.ds((base + k) * _CH, _CH)]

        def d_row(k):
            return ei_hbm.at[pl.ds(e_off + (base + k) * _CH, _CH)]

        def g_wait(i):
            pltpu.make_async_copy(
                hs_hbm.at[sidx.at[i % 4]], rows.at[i % 3], semg).wait()

        def s_drain(i):
            pltpu.make_async_copy(
                rows.at[i % 3], acc.at[didx.at[i % 3]], sems).wait()

        for k in range(4):
            pltpu.async_copy(s_row(k), sidx.at[k], semis)
        for k in range(2):
            pltpu.async_copy(d_row(k), didx.at[k], semid)
        for k in range(2):
            pltpu.make_async_copy(s_row(0), sidx.at[k], semis).wait()
            pltpu.async_copy(hs_hbm.at[sidx.at[k]], rows.at[k], semg)

        def body(i, carry):
            pltpu.make_async_copy(d_row(0), didx.at[i % 3], semid).wait()
            g_wait(i)
            pltpu.async_copy(rows.at[i % 3], acc.at[didx.at[i % 3]], sems,
                             add=True)

            @pl.when(i + 4 < n)
            def _():
                pltpu.async_copy(s_row(i + 4), sidx.at[i % 4], semis)

            @pl.when(i >= 1)
            def _():
                s_drain(i - 1)

            @pl.when(i + 2 < n)
            def _():
                pltpu.async_copy(d_row(i + 2), didx.at[(i + 2) % 3], semid)
                pltpu.make_async_copy(s_row(0), sidx.at[(i + 2) % 4],
                                      semis).wait()
                pltpu.async_copy(hs_hbm.at[sidx.at[(i + 2) % 4]],
                                 rows.at[(i + 2) % 3], semg)

            return carry

        lax.fori_loop(0, n, body, 0)
        s_drain(n - 1)
        plsc.subcore_barrier()
        pltpu.sync_copy(acc.at[pl.ds(s * rpt, rpt)],
                        out_hbm.at[c, pl.ds(s * rpt, rpt)])

    return seg_kernel


def _dinv_from_parts(dp):
    deg = dp[0, :, 0] + dp[1, :, 0] + 1.0
    return lax.rsqrt(deg)


def _stage1_body(x_ref, wp_ref, bp_ref, w_ref, degp_ref, hs_ref):
    dinv = _dinv_from_parts(degp_ref[...])
    x1 = jnp.maximum(
        jnp.dot(x_ref[...], wp_ref[...], preferred_element_type=jnp.float32)
        + bp_ref[...][None, :], 0.0)
    h = jnp.dot(x1, w_ref[...], preferred_element_type=jnp.float32)
    hs_ref[...] = dinv[:, None] * h


def _mid_body(sp_ref, hs_ref, degp_ref, b1_ref, wp2_ref, bp2_ref, w2_ref, hs2_ref):
    dinv = _dinv_from_parts(degp_ref[...])
    sp = sp_ref[...]
    agg = sp[0] + sp[1] + hs_ref[...]
    out1 = jnp.maximum(dinv[:, None] * agg + b1_ref[...][None, :], 0.0)
    x2 = jnp.maximum(
        jnp.dot(out1, wp2_ref[...], preferred_element_type=jnp.float32)
        + bp2_ref[...][None, :], 0.0)
    hs2_ref[...] = dinv[:, None] * jnp.dot(
        x2, w2_ref[...], preferred_element_type=jnp.float32)


def _fin_body(sp_ref, hs_ref, degp_ref, b2_ref, out_ref):
    dinv = _dinv_from_parts(degp_ref[...])
    sp = sp_ref[...]
    agg = sp[0] + sp[1] + hs_ref[...]
    out_ref[...] = jnp.maximum(dinv[:, None] * agg + b2_ref[...][None, :], 0.0)


def _row_spec(br):
    return pl.BlockSpec((br, _D), lambda i: (i, 0))


def _full_mat():
    return pl.BlockSpec((_D, _D), lambda i: (0, 0))


def _full_vec():
    return pl.BlockSpec((_D,), lambda i: (0,))


def _parts_spec(br, minor):
    return pl.BlockSpec((_NC, br, minor), lambda i: (0, i, 0))


def _tc_stage1(x, Wp1, bp1, W1, degp, br):
    n = x.shape[0]
    return pl.pallas_call(
        _stage1_body,
        grid=(n // br,),
        in_specs=[_row_spec(br), _full_mat(), _full_vec(), _full_mat(),
                  _parts_spec(br, _D)],
        out_specs=_row_spec(br),
        out_shape=jax.ShapeDtypeStruct((n, _D), jnp.float32),
    )(x, Wp1, bp1, W1, degp)


def _tc_mid(s1, hs1, degp, b1, Wp2, bp2, W2, br):
    n = hs1.shape[0]
    return pl.pallas_call(
        _mid_body,
        grid=(n // br,),
        in_specs=[_parts_spec(br, _D), _row_spec(br), _parts_spec(br, _D),
                  _full_vec(), _full_mat(), _full_vec(), _full_mat()],
        out_specs=_row_spec(br),
        out_shape=jax.ShapeDtypeStruct((n, _D), jnp.float32),
    )(s1, hs1, degp, b1, Wp2, bp2, W2)


def _tc_fin(s2, hs2, degp, b2, br):
    n = hs2.shape[0]
    return pl.pallas_call(
        _fin_body,
        grid=(n // br,),
        in_specs=[_parts_spec(br, _D), _row_spec(br), _parts_spec(br, _D),
                  _full_vec()],
        out_specs=_row_spec(br),
        out_shape=jax.ShapeDtypeStruct((n, _D), jnp.float32),
    )(s2, hs2, degp, b2)


def kernel(x, edge_index, Wp1, bp1, W1, b1, Wp2, bp2, W2, b2):
    n = x.shape[0]
    e = edge_index.shape[1]
    acc_rows = math.ceil((n + 1) / 128) * 128
    rpt = acc_rows // _NS
    br = 2000 if n % 2000 == 0 else 8 * math.gcd(n // 8, 1000)

    if e % _CH == 0:
        ei1d = edge_index.reshape(-1)
        e_off = e
    else:
        pad = _CH - e % _CH
        pi = jnp.arange(pad, dtype=edge_index.dtype)
        ei1d = jnp.concatenate([edge_index[0], pi % n,
                                edge_index[1], n + pi % (acc_rows - n)])
        e_off = e + pad
    e_chunks = e_off // _CH

    ones_rows = jnp.ones((_CH, _D), jnp.float32)
    zseg = jnp.zeros((rpt, _D), jnp.float32)

    deg_k = _make_deg_kernel(e_chunks, e_off, acc_rows)
    seg_k = _make_seg_kernel(e_chunks, e_off, acc_rows)

    degp = deg_k(ei1d, ones_rows, zseg)
    hs1 = _tc_stage1(x, Wp1, bp1, W1, degp, br)
    s1 = seg_k(hs1, ei1d, zseg)
    hs2 = _tc_mid(s1, hs1, degp, b1, Wp2, bp2, W2, br)
    s2 = seg_k(hs2, ei1d, zseg)
    return _tc_fin(s2, hs2, degp, b2, br)
